# Initial kernel scaffold; baseline (speedup 1.0000x reference)
import jax
import jax.numpy as jnp
from jax.experimental import pallas as pl


def kernel(inputs, uid_table, uid_bias_table, aid_table, aid_bias_table):
    raise NotImplementedError("write your pallas kernel here")


# speedup vs baseline: 1.6817x; 1.6817x over previous
import functools

import jax
import jax.numpy as jnp
from jax import lax
from jax.experimental import pallas as pl
from jax.experimental.pallas import tpu as pltpu
from jax.experimental.pallas import tpu_sc as plsc

B = 16384
D = 128
LANES = 16
NC = 2
NS = 16
NW = NC * NS
BPW = B // NW
CHUNK = 128
NCH = BPW // CHUNK
EPV = D // LANES


def _sc_body(uidx_hbm, aidx_hbm, ut_hbm, ubias_hbm, at_hbm, abias_hbm,
             part_out, ub_out, ab_out,
             uidx_v, aidx_v, u0, u1, a0, a1, ubv, abv, accv,
             sem0, sem1, semb):
    wid = lax.axis_index("s") * NC + lax.axis_index("c")
    pltpu.sync_copy(uidx_hbm.at[wid], uidx_v)
    pltpu.sync_copy(aidx_hbm.at[wid], aidx_v)

    bias_copies = []
    for ch in range(NCH):
        bias_copies.append(
            pltpu.async_copy(ubias_hbm.at[uidx_v.at[ch]], ubv.at[ch], semb))
        bias_copies.append(
            pltpu.async_copy(abias_hbm.at[aidx_v.at[ch]], abv.at[ch], semb))

    ubufs = (u0, u1)
    abufs = (a0, a1)
    sems = (sem0, sem1)

    def fire(ch):
        return (pltpu.async_copy(ut_hbm.at[uidx_v.at[ch]], ubufs[ch % 2],
                                 sems[ch % 2]),
                pltpu.async_copy(at_hbm.at[aidx_v.at[ch]], abufs[ch % 2],
                                 sems[ch % 2]))

    pending = {0: fire(0)}
    acc = jnp.zeros((LANES,), jnp.float32)
    for ch in range(NCH):
        if ch + 1 < NCH:
            pending[ch + 1] = fire(ch + 1)
        cu, ca = pending.pop(ch)
        cu.wait()
        ca.wait()
        ubuf = ubufs[ch % 2]
        abuf = abufs[ch % 2]

        def row_body(r, acc, ubuf=ubuf, abuf=abuf):
            for e in range(EPV):
                acc = acc + (ubuf[r, pl.ds(e * LANES, LANES)] *
                             abuf[r, pl.ds(e * LANES, LANES)])
            return acc

        acc = lax.fori_loop(0, CHUNK, row_body, acc)

    accv[...] = acc
    pltpu.sync_copy(accv, part_out.at[wid])
    for c in bias_copies:
        c.wait()
    pltpu.sync_copy(ubv, ub_out.at[wid])
    pltpu.sync_copy(abv, ab_out.at[wid])


_sc_call = functools.partial(
    pl.kernel,
    mesh=plsc.VectorSubcoreMesh(core_axis_name="c", subcore_axis_name="s"),
    out_type=[
        jax.ShapeDtypeStruct((NW, LANES), jnp.float32),
        jax.ShapeDtypeStruct((NW, NCH, CHUNK), jnp.float32),
        jax.ShapeDtypeStruct((NW, NCH, CHUNK), jnp.float32),
    ],
    scratch_types=[
        pltpu.VMEM((NCH, CHUNK), jnp.int32),
        pltpu.VMEM((NCH, CHUNK), jnp.int32),
        pltpu.VMEM((CHUNK, D), jnp.float32),
        pltpu.VMEM((CHUNK, D), jnp.float32),
        pltpu.VMEM((CHUNK, D), jnp.float32),
        pltpu.VMEM((CHUNK, D), jnp.float32),
        pltpu.VMEM((NCH, CHUNK), jnp.float32),
        pltpu.VMEM((NCH, CHUNK), jnp.float32),
        pltpu.VMEM((LANES,), jnp.float32),
        pltpu.SemaphoreType.DMA,
        pltpu.SemaphoreType.DMA,
        pltpu.SemaphoreType.DMA,
    ],
)(_sc_body)


def _combine(part_ref, ub_ref, ab_ref, o_ref):
    s = jnp.sum(part_ref[...])
    o_ref[...] = jax.nn.sigmoid(ub_ref[...] + ab_ref[...] + s)


def kernel(inputs, uid_table, uid_bias_table, aid_table, aid_bias_table):
    idx = inputs.astype(jnp.int32)
    uidx = idx[:, 0].reshape(NW, NCH, CHUNK)
    aidx = idx[:, 1].reshape(NW, NCH, CHUNK)
    ub1 = uid_bias_table.reshape(-1)
    ab1 = aid_bias_table.reshape(-1)

    part, ubg, abg = _sc_call(uidx, aidx, uid_table, ub1, aid_table, ab1)

    out = pl.pallas_call(
        _combine,
        out_shape=jax.ShapeDtypeStruct((B // D, D), jnp.float32),
    )(part, ubg.reshape(B // D, D), abg.reshape(B // D, D))
    return out.reshape(B, 1)
